# baseline (speedup 1.0000x reference)
import functools

import jax
import jax.numpy as jnp
from jax import lax
from jax.experimental import pallas as pl
from jax.experimental.pallas import tpu as pltpu
from jax.experimental.pallas import tpu_sc as plsc

F32 = jnp.float32
N = 10000
E = 320000
L = 128
DE = 16

NC = 2
NS = 16
NW = NC * NS

_BE = 3200
_BN = 2000

_PARTS = (102400, 102400, 115200)
_OFFS = (0, 102400, 204800)
_NP = len(_PARTS)


def _ln_relu(h, ls, lb):
    mu = jnp.mean(h, axis=-1, keepdims=True)
    d = h - mu
    var = jnp.mean(d * d, axis=-1, keepdims=True)
    return jnp.maximum(d * lax.rsqrt(var + 1e-6) * ls + lb, 0.0)


def _dense(x, w, b, block_rows):
    rows, k = x.shape
    out_cols = w.shape[1]

    def body(x_ref, w_ref, b_ref, o_ref):
        o_ref[...] = jnp.dot(x_ref[...], w_ref[...],
                             preferred_element_type=F32) + b_ref[...]

    return pl.pallas_call(
        body,
        grid=(rows // block_rows,),
        in_specs=[pl.BlockSpec((block_rows, k), lambda i: (i, 0)),
                  pl.BlockSpec((k, out_cols), lambda i: (0, 0)),
                  pl.BlockSpec((1, out_cols), lambda i: (0, 0))],
        out_specs=pl.BlockSpec((block_rows, out_cols), lambda i: (i, 0)),
        out_shape=jax.ShapeDtypeStruct((rows, out_cols), F32),
    )(x, w, b)


def _proj(n, ew, eb, g, step):
    nb = N // _BN

    def body(n_ref, w_ref, eb_ref, g_ref, ps_ref, pr_ref, gv_ref):
        x = n_ref[...]
        ps_ref[...] = jnp.dot(x, w_ref[0, L:2 * L, :],
                              preferred_element_type=F32)
        pr_ref[...] = jnp.dot(x, w_ref[0, 2 * L:3 * L, :],
                              preferred_element_type=F32)

        @pl.when(pl.program_id(0) == 0)
        def _():
            gv_ref[...] = jnp.dot(g_ref[...], w_ref[0, 3 * L:4 * L, :],
                                  preferred_element_type=F32) + eb_ref[0]

    full = lambda i: (0, 0)
    stepw = lambda i: (step, 0, 0)
    stepb = lambda i: (step, 0, 0)
    return pl.pallas_call(
        body,
        grid=(nb,),
        in_specs=[pl.BlockSpec((_BN, L), lambda i: (i, 0)),
                  pl.BlockSpec((1, 4 * L, L), stepw),
                  pl.BlockSpec((1, 1, L), stepb),
                  pl.BlockSpec((1, L), full)],
        out_specs=[pl.BlockSpec((_BN, L), lambda i: (i, 0)),
                   pl.BlockSpec((_BN, L), lambda i: (i, 0)),
                   pl.BlockSpec((1, L), full)],
        out_shape=[jax.ShapeDtypeStruct((N, L), F32),
                   jax.ShapeDtypeStruct((N, L), F32),
                   jax.ShapeDtypeStruct((1, L), F32)],
    )(n, ew, eb, g)


def _edge_mlp(e, sum_rows, part, ew, gvec, els, elb, step):
    nb = _PARTS[part] // _BE

    def body(e_ref, s_ref, w_ref, gv_ref, ls_ref, lb_ref, o_ref, agg_ref):
        h = (jnp.dot(e_ref[...], w_ref[0, 0:L, :],
                     preferred_element_type=F32)
             + s_ref[...] + gv_ref[...])
        a = _ln_relu(h, ls_ref[0], lb_ref[0])
        o_ref[...] = a

        @pl.when(pl.program_id(0) == 0)
        def _():
            agg_ref[...] = jnp.zeros_like(agg_ref)

        agg_ref[...] += jnp.sum(a, axis=0, keepdims=True)

    full = lambda i: (0, 0)
    blk = lambda i: (i, 0)
    return pl.pallas_call(
        body,
        grid=(nb,),
        in_specs=[pl.BlockSpec((_BE, L), blk),
                  pl.BlockSpec((_BE, L), blk),
                  pl.BlockSpec((1, 4 * L, L), lambda i: (step, 0, 0)),
                  pl.BlockSpec((1, L), full),
                  pl.BlockSpec((1, 1, L), lambda i: (step, 0, 0)),
                  pl.BlockSpec((1, 1, L), lambda i: (step, 0, 0))],
        out_specs=[pl.BlockSpec((_BE, L), blk),
                   pl.BlockSpec((1, L), full)],
        out_shape=[jax.ShapeDtypeStruct((_PARTS[part], L), F32),
                   jax.ShapeDtypeStruct((1, L), F32)],
    )(e, sum_rows, ew, gvec, els, elb)


def _edge_mlp0(efT, sum_rows, part, wemb, bemb, ew, gvec, els, elb):
    nb = _PARTS[part] // _BE
    off = _OFFS[part] // _BE

    def body(ef_ref, s_ref, wemb_ref, bemb_ref, w_ref, gv_ref,
             ls_ref, lb_ref, o_ref, agg_ref):
        t = lax.dot_general(ef_ref[...], wemb_ref[...],
                            (((0,), (0,)), ((), ())),
                            preferred_element_type=F32) + bemb_ref[...]
        h = (jnp.dot(t, w_ref[0, 0:L, :], preferred_element_type=F32)
             + s_ref[...] + gv_ref[...])
        a = _ln_relu(h, ls_ref[0], lb_ref[0])
        o_ref[...] = a

        @pl.when(pl.program_id(0) == 0)
        def _():
            agg_ref[...] = jnp.zeros_like(agg_ref)

        agg_ref[...] += jnp.sum(a, axis=0, keepdims=True)

    full = lambda i: (0, 0)
    blk = lambda i: (i, 0)
    return pl.pallas_call(
        body,
        grid=(nb,),
        in_specs=[pl.BlockSpec((DE, _BE), lambda i: (0, i + off)),
                  pl.BlockSpec((_BE, L), blk),
                  pl.BlockSpec((DE, L), full),
                  pl.BlockSpec((1, L), full),
                  pl.BlockSpec((1, 4 * L, L), lambda i: (0, 0, 0)),
                  pl.BlockSpec((1, L), full),
                  pl.BlockSpec((1, 1, L), lambda i: (0, 0, 0)),
                  pl.BlockSpec((1, 1, L), lambda i: (0, 0, 0))],
        out_specs=[pl.BlockSpec((_BE, L), blk),
                   pl.BlockSpec((1, L), full)],
        out_shape=[jax.ShapeDtypeStruct((_PARTS[part], L), F32),
                   jax.ShapeDtypeStruct((1, L), F32)],
    )(efT, sum_rows, wemb, bemb, ew, gvec, els, elb)


def _node_glob(n, parts, nw, nb_, nls, nlb, g, eaggs, gw, gb, gls, glb, step):
    nblk = N // _BN

    def body(n_ref, p0_ref, p1_ref, p2_ref, q0_ref, q1_ref, q2_ref,
             w_ref, b_ref, ls_ref, lb_ref, g_ref, ea0_ref, ea1_ref, ea2_ref,
             gw_ref, gb_ref, gls_ref, glb_ref, o_ref, na_ref, go_ref):
        s = p0_ref[0] + p1_ref[0] + p2_ref[0]
        r = q0_ref[0] + q1_ref[0] + q2_ref[0]
        h = (jnp.dot(n_ref[...], w_ref[0, 0:L, :],
                     preferred_element_type=F32)
             + jnp.dot(s, w_ref[0, L:2 * L, :], preferred_element_type=F32)
             + jnp.dot(r, w_ref[0, 2 * L:3 * L, :],
                       preferred_element_type=F32)
             + jnp.dot(g_ref[...], w_ref[0, 3 * L:4 * L, :],
                       preferred_element_type=F32)
             + b_ref[0])
        a = _ln_relu(h, ls_ref[0], lb_ref[0])
        o_ref[...] = a

        @pl.when(pl.program_id(0) == 0)
        def _():
            na_ref[...] = jnp.zeros_like(na_ref)

        na_ref[...] += jnp.sum(a, axis=0, keepdims=True)

        @pl.when(pl.program_id(0) == nblk - 1)
        def _():
            ea = ea0_ref[...] + ea1_ref[...] + ea2_ref[...]
            hg = (jnp.dot(na_ref[...], gw_ref[0, 0:L, :],
                          preferred_element_type=F32)
                  + jnp.dot(ea, gw_ref[0, L:2 * L, :],
                            preferred_element_type=F32)
                  + jnp.dot(g_ref[...], gw_ref[0, 2 * L:3 * L, :],
                            preferred_element_type=F32)
                  + gb_ref[0])
            go_ref[...] = _ln_relu(hg, gls_ref[0], glb_ref[0])

    full = lambda i: (0, 0)
    blk = lambda i: (i, 0)
    stepb = lambda i: (step, 0, 0)
    return pl.pallas_call(
        body,
        grid=(nblk,),
        in_specs=[pl.BlockSpec((_BN, L), blk)]
        + [pl.BlockSpec((1, _BN, L), lambda i: (0, i, 0))] * 3
        + [pl.BlockSpec((1, _BN, L), lambda i: (1, i, 0))] * 3
        + [pl.BlockSpec((1, 4 * L, L), lambda i: (step, 0, 0)),
           pl.BlockSpec((1, 1, L), stepb),
           pl.BlockSpec((1, 1, L), stepb),
           pl.BlockSpec((1, 1, L), stepb),
           pl.BlockSpec((1, L), full),
           pl.BlockSpec((1, L), full),
           pl.BlockSpec((1, L), full),
           pl.BlockSpec((1, L), full),
           pl.BlockSpec((1, 3 * L, L), lambda i: (step, 0, 0)),
           pl.BlockSpec((1, 1, L), stepb),
           pl.BlockSpec((1, 1, L), stepb),
           pl.BlockSpec((1, 1, L), stepb)],
        out_specs=[pl.BlockSpec((_BN, L), blk),
                   pl.BlockSpec((1, L), full),
                   pl.BlockSpec((1, L), full)],
        out_shape=[jax.ShapeDtypeStruct((N, L), F32),
                   jax.ShapeDtypeStruct((1, L), F32),
                   jax.ShapeDtypeStruct((1, L), F32)],
    )(n, parts[0], parts[1], parts[2], parts[0], parts[1], parts[2],
      nw, nb_, nls, nlb, g, eaggs[0], eaggs[1], eaggs[2],
      gw, gb, gls, glb)


_CG = 40
_NBG = 10

_MESH = plsc.VectorSubcoreMesh(core_axis_name="c", subcore_axis_name="s")


def _make_gather(ep, off):
    epw = ep // NW
    ngrp = epw // _CG // _NBG

    @functools.partial(
        pl.kernel,
        mesh=_MESH,
        out_type=jax.ShapeDtypeStruct((ep, L), F32),
        scratch_types=[pltpu.VMEM((epw,), jnp.int32),
                       pltpu.VMEM((epw,), jnp.int32),
                       pltpu.VMEM((_NBG, _CG, L), F32),
                       pltpu.VMEM((_NBG, _CG, L), F32),
                       pltpu.SemaphoreType.DMA((_NBG,)),
                       pltpu.SemaphoreType.DMA((_NBG,)),
                       pltpu.SemaphoreType.DMA((_NBG,))],
    )
    def gather(ps_hbm, pr_hbm, snd_hbm, rcv_hbm, sum_out,
               idx_s, idx_r, rows_s, rows_r, sem_gs, sem_gr, sem_ws):
        wid = lax.axis_index("s") * NC + lax.axis_index("c")
        base0 = pl.multiple_of(wid * epw, 8)
        ibase0 = pl.multiple_of(off + wid * epw, 8)

        pltpu.sync_copy(snd_hbm.at[pl.ds(ibase0, epw)], idx_s)
        pltpu.sync_copy(rcv_hbm.at[pl.ds(ibase0, epw)], idx_r)

        def start_gather(b, ci):
            o2 = pl.multiple_of(ci * _CG, 8)
            pltpu.async_copy(ps_hbm.at[idx_s.at[pl.ds(o2, _CG)]],
                             rows_s.at[b], sem_gs.at[b])
            pltpu.async_copy(pr_hbm.at[idx_r.at[pl.ds(o2, _CG)]],
                             rows_r.at[b], sem_gr.at[b])

        for b in range(_NBG):
            start_gather(b, b)

        def group(g, carry):
            for b in range(_NBG):
                ci = g * _NBG + b
                base = pl.multiple_of(base0 + ci * _CG, 8)
                pltpu.make_async_copy(ps_hbm.at[idx_s.at[pl.ds(0, _CG)]],
                                      rows_s.at[b], sem_gs.at[b]).wait()
                pltpu.make_async_copy(pr_hbm.at[idx_r.at[pl.ds(0, _CG)]],
                                      rows_r.at[b], sem_gr.at[b]).wait()

                def add_row(r2, c2):
                    for j in range(L // 16):
                        sl = pl.ds(j * 16, 16)
                        rows_s[b, r2, sl] = (rows_s[b, r2, sl]
                                             + rows_r[b, r2, sl])
                    return c2

                lax.fori_loop(0, _CG, add_row, 0)
                ws = pltpu.async_copy(rows_s.at[b],
                                      sum_out.at[pl.ds(base, _CG)],
                                      sem_ws.at[b])
                ws.wait()

                @pl.when(g < ngrp - 1)
                def _():
                    start_gather(b, ci + _NBG)
            return carry

        lax.fori_loop(0, ngrp, group, 0)

    return gather


_GATHERS = tuple(_make_gather(s, o) for s, o in zip(_PARTS, _OFFS))


_CS = 40
_NBS = 5


def _make_scatter(ep):
    ept = ep // NS
    ngrps = ept // _CS // _NBS

    @functools.partial(
        pl.kernel,
        mesh=_MESH,
        out_type=jax.ShapeDtypeStruct((2, N, L), F32),
        scratch_types=[pltpu.VMEM((_NBS, _CS), jnp.int32),
                       pltpu.VMEM((_NBS, _CS, L), F32),
                       pltpu.VMEM_SHARED((N, L), F32),
                       pltpu.SemaphoreType.DMA((_NBS,)),
                       pltpu.SemaphoreType.DMA((_NBS,))],
    )
    def scatter(e_hbm, idx2_hbm, z_hbm, out_hbm,
                idx_b, rows_v, acc, sem_ld, sem_ix):
        cid = lax.axis_index("c")
        sid = lax.axis_index("s")

        @pl.when(sid == 0)
        def _():
            pltpu.sync_copy(z_hbm, acc)

        base0 = pl.multiple_of(sid * ept, 8)
        ibase0 = pl.multiple_of(cid * ep + sid * ept, 8)
        plsc.subcore_barrier()

        def start_load(b, ci):
            base = pl.multiple_of(base0 + ci * _CS, 8)
            ibase = pl.multiple_of(ibase0 + ci * _CS, 8)
            pltpu.async_copy(idx2_hbm.at[pl.ds(ibase, _CS)], idx_b.at[b],
                             sem_ix.at[b])
            pltpu.async_copy(e_hbm.at[pl.ds(base, _CS)], rows_v.at[b],
                             sem_ld.at[b])

        for b in range(_NBS):
            start_load(b, b)

        def group(g, carry):
            for b in range(_NBS):
                ci = g * _NBS + b
                pltpu.make_async_copy(idx2_hbm.at[pl.ds(0, _CS)],
                                      idx_b.at[b], sem_ix.at[b]).wait()
                pltpu.make_async_copy(e_hbm.at[pl.ds(0, _CS)],
                                      rows_v.at[b], sem_ld.at[b]).wait()
                pltpu.sync_copy(rows_v.at[b], acc.at[idx_b.at[b]], add=True)

                @pl.when(g < ngrps - 1)
                def _():
                    start_load(b, ci + _NBS)
            return carry

        lax.fori_loop(0, ngrps, group, 0)
        plsc.subcore_barrier()

        @pl.when(sid == 0)
        def _():
            pltpu.sync_copy(acc, out_hbm.at[cid])

    return scatter


_SCATTERS = {ep: _make_scatter(ep) for ep in set(_PARTS)}


def kernel(nodes, edge_feats, senders, receivers,
           embed_node_W, embed_node_b, embed_edge_W, embed_edge_b,
           edge_W, edge_b, edge_ls, edge_lb,
           node_W, node_b, node_ls, node_lb,
           glob_W, glob_b, glob_ls, glob_lb,
           dec_W, dec_b):
    steps = edge_W.shape[0]
    row = lambda v: v.reshape(1, -1)
    r3 = lambda v: v.reshape(steps, 1, -1)
    edge_b, edge_ls, edge_lb = r3(edge_b), r3(edge_ls), r3(edge_lb)
    node_b, node_ls, node_lb = r3(node_b), r3(node_ls), r3(node_lb)
    glob_b, glob_ls, glob_lb = r3(glob_b), r3(glob_ls), r3(glob_lb)

    n = _dense(nodes, embed_node_W, row(embed_node_b), _BN)
    efT = jnp.transpose(edge_feats)
    g = jnp.zeros((1, L), F32)
    zeros_n = jnp.zeros((N, L), F32)
    snd = senders.astype(jnp.int32)
    rcv = receivers.astype(jnp.int32)
    idx2p = tuple(jnp.concatenate([snd[o:o + s], rcv[o:o + s]])
                  for s, o in zip(_PARTS, _OFFS))

    e = [None] * _NP
    for i in range(steps):
        ps, pr, gvec = _proj(n, edge_W, edge_b, g, i)
        sums = [_GATHERS[p](ps, pr, snd, rcv) for p in range(_NP)]
        eaggs = [None] * _NP
        parts = [None] * _NP
        for p in range(_NP):
            if i == 0:
                e[p], eaggs[p] = _edge_mlp0(
                    efT, sums[p], p, embed_edge_W, row(embed_edge_b),
                    edge_W, gvec, edge_ls, edge_lb)
            else:
                e[p], eaggs[p] = _edge_mlp(
                    e[p], sums[p], p, edge_W, gvec, edge_ls, edge_lb, i)
            parts[p] = _SCATTERS[_PARTS[p]](e[p], idx2p[p], zeros_n)
        n, _nagg, g = _node_glob(
            n, parts, node_W, node_b, node_ls, node_lb,
            g, eaggs, glob_W, glob_b, glob_ls, glob_lb, i)

    return _dense(g, dec_W, row(dec_b), 1)


# speedup vs baseline: 1.0112x; 1.0112x over previous
import functools

import jax
import jax.numpy as jnp
from jax import lax
from jax.experimental import pallas as pl
from jax.experimental.pallas import tpu as pltpu
from jax.experimental.pallas import tpu_sc as plsc

F32 = jnp.float32
N = 10000
E = 320000
L = 128
DE = 16

NC = 2
NS = 16
NW = NC * NS

_BE = 3200
_BN = 2000

_PARTS = (105600, 105600, 108800)
_OFFS = (0, 105600, 211200)
_NP = len(_PARTS)


def _ln_relu(h, ls, lb):
    mu = jnp.mean(h, axis=-1, keepdims=True)
    d = h - mu
    var = jnp.mean(d * d, axis=-1, keepdims=True)
    return jnp.maximum(d * lax.rsqrt(var + 1e-6) * ls + lb, 0.0)


def _dense(x, w, b, block_rows):
    rows, k = x.shape
    out_cols = w.shape[1]

    def body(x_ref, w_ref, b_ref, o_ref):
        o_ref[...] = jnp.dot(x_ref[...], w_ref[...],
                             preferred_element_type=F32) + b_ref[...]

    return pl.pallas_call(
        body,
        grid=(rows // block_rows,),
        in_specs=[pl.BlockSpec((block_rows, k), lambda i: (i, 0)),
                  pl.BlockSpec((k, out_cols), lambda i: (0, 0)),
                  pl.BlockSpec((1, out_cols), lambda i: (0, 0))],
        out_specs=pl.BlockSpec((block_rows, out_cols), lambda i: (i, 0)),
        out_shape=jax.ShapeDtypeStruct((rows, out_cols), F32),
    )(x, w, b)


def _proj(n, ew, eb, g, step):
    nb = N // _BN

    def body(n_ref, w_ref, eb_ref, g_ref, ps_ref, pr_ref, gv_ref):
        x = n_ref[...]
        ps_ref[...] = jnp.dot(x, w_ref[0, L:2 * L, :],
                              preferred_element_type=F32)
        pr_ref[...] = jnp.dot(x, w_ref[0, 2 * L:3 * L, :],
                              preferred_element_type=F32)

        @pl.when(pl.program_id(0) == 0)
        def _():
            gv_ref[...] = jnp.dot(g_ref[...], w_ref[0, 3 * L:4 * L, :],
                                  preferred_element_type=F32) + eb_ref[0]

    full = lambda i: (0, 0)
    stepw = lambda i: (step, 0, 0)
    stepb = lambda i: (step, 0, 0)
    return pl.pallas_call(
        body,
        grid=(nb,),
        in_specs=[pl.BlockSpec((_BN, L), lambda i: (i, 0)),
                  pl.BlockSpec((1, 4 * L, L), stepw),
                  pl.BlockSpec((1, 1, L), stepb),
                  pl.BlockSpec((1, L), full)],
        out_specs=[pl.BlockSpec((_BN, L), lambda i: (i, 0)),
                   pl.BlockSpec((_BN, L), lambda i: (i, 0)),
                   pl.BlockSpec((1, L), full)],
        out_shape=[jax.ShapeDtypeStruct((N, L), F32),
                   jax.ShapeDtypeStruct((N, L), F32),
                   jax.ShapeDtypeStruct((1, L), F32)],
    )(n, ew, eb, g)


def _edge_mlp(e, sum_rows, part, ew, gvec, els, elb, step):
    nb = _PARTS[part] // _BE
    off = _OFFS[part] // _BE

    def body(e_ref, s_ref, w_ref, gv_ref, ls_ref, lb_ref, o_ref, agg_ref):
        h = (jnp.dot(e_ref[...], w_ref[0, 0:L, :],
                     preferred_element_type=F32)
             + s_ref[...] + gv_ref[...])
        a = _ln_relu(h, ls_ref[0], lb_ref[0])
        o_ref[...] = a

        @pl.when(pl.program_id(0) == 0)
        def _():
            agg_ref[...] = jnp.zeros_like(agg_ref)

        agg_ref[...] += jnp.sum(a, axis=0, keepdims=True)

    full = lambda i: (0, 0)
    blk = lambda i: (i, 0)
    return pl.pallas_call(
        body,
        grid=(nb,),
        in_specs=[pl.BlockSpec((_BE, L), blk),
                  pl.BlockSpec((_BE, L), lambda i: (i + off, 0)),
                  pl.BlockSpec((1, 4 * L, L), lambda i: (step, 0, 0)),
                  pl.BlockSpec((1, L), full),
                  pl.BlockSpec((1, 1, L), lambda i: (step, 0, 0)),
                  pl.BlockSpec((1, 1, L), lambda i: (step, 0, 0))],
        out_specs=[pl.BlockSpec((_BE, L), blk),
                   pl.BlockSpec((1, L), full)],
        out_shape=[jax.ShapeDtypeStruct((_PARTS[part], L), F32),
                   jax.ShapeDtypeStruct((1, L), F32)],
    )(e, sum_rows, ew, gvec, els, elb)


def _edge_mlp0(efT, sum_rows, part, wemb, bemb, ew, gvec, els, elb):
    nb = _PARTS[part] // _BE
    off = _OFFS[part] // _BE

    def body(ef_ref, s_ref, wemb_ref, bemb_ref, w_ref, gv_ref,
             ls_ref, lb_ref, o_ref, agg_ref):
        t = lax.dot_general(ef_ref[...], wemb_ref[...],
                            (((0,), (0,)), ((), ())),
                            preferred_element_type=F32) + bemb_ref[...]
        h = (jnp.dot(t, w_ref[0, 0:L, :], preferred_element_type=F32)
             + s_ref[...] + gv_ref[...])
        a = _ln_relu(h, ls_ref[0], lb_ref[0])
        o_ref[...] = a

        @pl.when(pl.program_id(0) == 0)
        def _():
            agg_ref[...] = jnp.zeros_like(agg_ref)

        agg_ref[...] += jnp.sum(a, axis=0, keepdims=True)

    full = lambda i: (0, 0)
    blk = lambda i: (i, 0)
    return pl.pallas_call(
        body,
        grid=(nb,),
        in_specs=[pl.BlockSpec((DE, _BE), lambda i: (0, i + off)),
                  pl.BlockSpec((_BE, L), lambda i: (i + off, 0)),
                  pl.BlockSpec((DE, L), full),
                  pl.BlockSpec((1, L), full),
                  pl.BlockSpec((1, 4 * L, L), lambda i: (0, 0, 0)),
                  pl.BlockSpec((1, L), full),
                  pl.BlockSpec((1, 1, L), lambda i: (0, 0, 0)),
                  pl.BlockSpec((1, 1, L), lambda i: (0, 0, 0))],
        out_specs=[pl.BlockSpec((_BE, L), blk),
                   pl.BlockSpec((1, L), full)],
        out_shape=[jax.ShapeDtypeStruct((_PARTS[part], L), F32),
                   jax.ShapeDtypeStruct((1, L), F32)],
    )(efT, sum_rows, wemb, bemb, ew, gvec, els, elb)


def _node_glob(n, parts, nw, nb_, nls, nlb, g, eaggs, gw, gb, gls, glb, step):
    nblk = N // _BN

    def body(n_ref, p0_ref, p1_ref, p2_ref, q0_ref, q1_ref, q2_ref,
             w_ref, b_ref, ls_ref, lb_ref, g_ref, ea0_ref, ea1_ref, ea2_ref,
             gw_ref, gb_ref, gls_ref, glb_ref, o_ref, na_ref, go_ref):
        s = p0_ref[0] + p1_ref[0] + p2_ref[0]
        r = q0_ref[0] + q1_ref[0] + q2_ref[0]
        h = (jnp.dot(n_ref[...], w_ref[0, 0:L, :],
                     preferred_element_type=F32)
             + jnp.dot(s, w_ref[0, L:2 * L, :], preferred_element_type=F32)
             + jnp.dot(r, w_ref[0, 2 * L:3 * L, :],
                       preferred_element_type=F32)
             + jnp.dot(g_ref[...], w_ref[0, 3 * L:4 * L, :],
                       preferred_element_type=F32)
             + b_ref[0])
        a = _ln_relu(h, ls_ref[0], lb_ref[0])
        o_ref[...] = a

        @pl.when(pl.program_id(0) == 0)
        def _():
            na_ref[...] = jnp.zeros_like(na_ref)

        na_ref[...] += jnp.sum(a, axis=0, keepdims=True)

        @pl.when(pl.program_id(0) == nblk - 1)
        def _():
            ea = ea0_ref[...] + ea1_ref[...] + ea2_ref[...]
            hg = (jnp.dot(na_ref[...], gw_ref[0, 0:L, :],
                          preferred_element_type=F32)
                  + jnp.dot(ea, gw_ref[0, L:2 * L, :],
                            preferred_element_type=F32)
                  + jnp.dot(g_ref[...], gw_ref[0, 2 * L:3 * L, :],
                            preferred_element_type=F32)
                  + gb_ref[0])
            go_ref[...] = _ln_relu(hg, gls_ref[0], glb_ref[0])

    full = lambda i: (0, 0)
    blk = lambda i: (i, 0)
    stepb = lambda i: (step, 0, 0)
    return pl.pallas_call(
        body,
        grid=(nblk,),
        in_specs=[pl.BlockSpec((_BN, L), blk)]
        + [pl.BlockSpec((1, _BN, L), lambda i: (0, i, 0))] * 3
        + [pl.BlockSpec((1, _BN, L), lambda i: (1, i, 0))] * 3
        + [pl.BlockSpec((1, 4 * L, L), lambda i: (step, 0, 0)),
           pl.BlockSpec((1, 1, L), stepb),
           pl.BlockSpec((1, 1, L), stepb),
           pl.BlockSpec((1, 1, L), stepb),
           pl.BlockSpec((1, L), full),
           pl.BlockSpec((1, L), full),
           pl.BlockSpec((1, L), full),
           pl.BlockSpec((1, L), full),
           pl.BlockSpec((1, 3 * L, L), lambda i: (step, 0, 0)),
           pl.BlockSpec((1, 1, L), stepb),
           pl.BlockSpec((1, 1, L), stepb),
           pl.BlockSpec((1, 1, L), stepb)],
        out_specs=[pl.BlockSpec((_BN, L), blk),
                   pl.BlockSpec((1, L), full),
                   pl.BlockSpec((1, L), full)],
        out_shape=[jax.ShapeDtypeStruct((N, L), F32),
                   jax.ShapeDtypeStruct((1, L), F32),
                   jax.ShapeDtypeStruct((1, L), F32)],
    )(n, parts[0], parts[1], parts[2], parts[0], parts[1], parts[2],
      nw, nb_, nls, nlb, g, eaggs[0], eaggs[1], eaggs[2],
      gw, gb, gls, glb)


_EPW = E // NW
_CG = 80
_NBG = 5
_NGRP = _EPW // _CG // _NBG

_CS = 40
_NBS = 5

_MESH = plsc.VectorSubcoreMesh(core_axis_name="c", subcore_axis_name="s")


@functools.partial(
    pl.kernel,
    mesh=_MESH,
    out_type=jax.ShapeDtypeStruct((E, L), F32),
    scratch_types=[pltpu.VMEM((_EPW,), jnp.int32),
                   pltpu.VMEM((_EPW,), jnp.int32),
                   pltpu.VMEM((_NBG, _CG, L), F32),
                   pltpu.VMEM((_NBG, _CG, L), F32),
                   pltpu.SemaphoreType.DMA((_NBG,)),
                   pltpu.SemaphoreType.DMA((_NBG,)),
                   pltpu.SemaphoreType.DMA((_NBG,))],
)
def _sc_gather(ps_hbm, pr_hbm, snd_hbm, rcv_hbm, sum_out,
               idx_s, idx_r, rows_s, rows_r, sem_gs, sem_gr, sem_ws):
    wid = lax.axis_index("s") * NC + lax.axis_index("c")
    base0 = pl.multiple_of(wid * _EPW, 8)

    pltpu.sync_copy(snd_hbm.at[pl.ds(base0, _EPW)], idx_s)
    pltpu.sync_copy(rcv_hbm.at[pl.ds(base0, _EPW)], idx_r)

    def start_gather(b, ci):
        off = pl.multiple_of(ci * _CG, 8)
        pltpu.async_copy(ps_hbm.at[idx_s.at[pl.ds(off, _CG)]],
                         rows_s.at[b], sem_gs.at[b])
        pltpu.async_copy(pr_hbm.at[idx_r.at[pl.ds(off, _CG)]],
                         rows_r.at[b], sem_gr.at[b])

    for b in range(_NBG):
        start_gather(b, b)

    def group(g, carry):
        for b in range(_NBG):
            ci = g * _NBG + b
            base = pl.multiple_of(base0 + ci * _CG, 8)
            pltpu.make_async_copy(ps_hbm.at[idx_s.at[pl.ds(0, _CG)]],
                                  rows_s.at[b], sem_gs.at[b]).wait()
            pltpu.make_async_copy(pr_hbm.at[idx_r.at[pl.ds(0, _CG)]],
                                  rows_r.at[b], sem_gr.at[b]).wait()

            def add_row(r2, c2):
                for u in range(2):
                    for j in range(L // 16):
                        sl = pl.ds(j * 16, 16)
                        rows_s[b, 2 * r2 + u, sl] = (
                            rows_s[b, 2 * r2 + u, sl]
                            + rows_r[b, 2 * r2 + u, sl])
                return c2

            lax.fori_loop(0, _CG // 2, add_row, 0)
            ws = pltpu.async_copy(rows_s.at[b], sum_out.at[pl.ds(base, _CG)],
                                  sem_ws.at[b])

            @pl.when(g < _NGRP - 1)
            def _():
                off2 = pl.multiple_of((ci + _NBG) * _CG, 8)
                pltpu.async_copy(pr_hbm.at[idx_r.at[pl.ds(off2, _CG)]],
                                 rows_r.at[b], sem_gr.at[b])

            ws.wait()

            @pl.when(g < _NGRP - 1)
            def _():
                off2 = pl.multiple_of((ci + _NBG) * _CG, 8)
                pltpu.async_copy(ps_hbm.at[idx_s.at[pl.ds(off2, _CG)]],
                                 rows_s.at[b], sem_gs.at[b])
        return carry

    lax.fori_loop(0, _NGRP, group, 0)


def _make_scatter(ep):
    ept = ep // NS
    ngrps = ept // _CS // _NBS

    @functools.partial(
        pl.kernel,
        mesh=_MESH,
        out_type=jax.ShapeDtypeStruct((2, N, L), F32),
        scratch_types=[pltpu.VMEM((_NBS, _CS), jnp.int32),
                       pltpu.VMEM((_NBS, _CS, L), F32),
                       pltpu.VMEM_SHARED((N, L), F32),
                       pltpu.SemaphoreType.DMA((_NBS,)),
                       pltpu.SemaphoreType.DMA((_NBS,))],
    )
    def scatter(e_hbm, idx2_hbm, z_hbm, out_hbm,
                idx_b, rows_v, acc, sem_ld, sem_ix):
        cid = lax.axis_index("c")
        sid = lax.axis_index("s")

        @pl.when(sid == 0)
        def _():
            pltpu.sync_copy(z_hbm, acc)

        base0 = pl.multiple_of(sid * ept, 8)
        ibase0 = pl.multiple_of(cid * ep + sid * ept, 8)
        plsc.subcore_barrier()

        def start_load(b, ci):
            base = pl.multiple_of(base0 + ci * _CS, 8)
            ibase = pl.multiple_of(ibase0 + ci * _CS, 8)
            pltpu.async_copy(idx2_hbm.at[pl.ds(ibase, _CS)], idx_b.at[b],
                             sem_ix.at[b])
            pltpu.async_copy(e_hbm.at[pl.ds(base, _CS)], rows_v.at[b],
                             sem_ld.at[b])

        for b in range(_NBS):
            start_load(b, b)

        def group(g, carry):
            for b in range(_NBS):
                ci = g * _NBS + b
                pltpu.make_async_copy(idx2_hbm.at[pl.ds(0, _CS)],
                                      idx_b.at[b], sem_ix.at[b]).wait()
                pltpu.make_async_copy(e_hbm.at[pl.ds(0, _CS)],
                                      rows_v.at[b], sem_ld.at[b]).wait()
                pltpu.sync_copy(rows_v.at[b], acc.at[idx_b.at[b]], add=True)

                @pl.when(g < ngrps - 1)
                def _():
                    start_load(b, ci + _NBS)
            return carry

        lax.fori_loop(0, ngrps, group, 0)
        plsc.subcore_barrier()

        @pl.when(sid == 0)
        def _():
            pltpu.sync_copy(acc, out_hbm.at[cid])

    return scatter


_SCATTERS = {ep: _make_scatter(ep) for ep in set(_PARTS)}


def kernel(nodes, edge_feats, senders, receivers,
           embed_node_W, embed_node_b, embed_edge_W, embed_edge_b,
           edge_W, edge_b, edge_ls, edge_lb,
           node_W, node_b, node_ls, node_lb,
           glob_W, glob_b, glob_ls, glob_lb,
           dec_W, dec_b):
    steps = edge_W.shape[0]
    row = lambda v: v.reshape(1, -1)
    r3 = lambda v: v.reshape(steps, 1, -1)
    edge_b, edge_ls, edge_lb = r3(edge_b), r3(edge_ls), r3(edge_lb)
    node_b, node_ls, node_lb = r3(node_b), r3(node_ls), r3(node_lb)
    glob_b, glob_ls, glob_lb = r3(glob_b), r3(glob_ls), r3(glob_lb)

    n = _dense(nodes, embed_node_W, row(embed_node_b), _BN)
    efT = jnp.transpose(edge_feats)
    g = jnp.zeros((1, L), F32)
    zeros_n = jnp.zeros((N, L), F32)
    snd = senders.astype(jnp.int32)
    rcv = receivers.astype(jnp.int32)
    idx2p = tuple(jnp.concatenate([snd[o:o + s], rcv[o:o + s]])
                  for s, o in zip(_PARTS, _OFFS))

    e = [None] * _NP
    for i in range(steps):
        ps, pr, gvec = _proj(n, edge_W, edge_b, g, i)
        sum_rows = _sc_gather(ps, pr, snd, rcv)
        eaggs = [None] * _NP
        parts = [None] * _NP
        for p in range(_NP):
            if i == 0:
                e[p], eaggs[p] = _edge_mlp0(
                    efT, sum_rows, p, embed_edge_W, row(embed_edge_b),
                    edge_W, gvec, edge_ls, edge_lb)
            else:
                e[p], eaggs[p] = _edge_mlp(
                    e[p], sum_rows, p, edge_W, gvec, edge_ls, edge_lb, i)
            parts[p] = _SCATTERS[_PARTS[p]](e[p], idx2p[p], zeros_n)
        n, _nagg, g = _node_glob(
            n, parts, node_W, node_b, node_ls, node_lb,
            g, eaggs, glob_W, glob_b, glob_ls, glob_lb, i)

    return _dense(g, dec_W, row(dec_b), 1)


# speedup vs baseline: 1.0135x; 1.0023x over previous
import functools

import jax
import jax.numpy as jnp
from jax import lax
from jax.experimental import pallas as pl
from jax.experimental.pallas import tpu as pltpu
from jax.experimental.pallas import tpu_sc as plsc

F32 = jnp.float32
N = 10000
E = 320000
L = 128
DE = 16

NC = 2
NS = 16
NW = NC * NS

_BE = 3200
_BN = 2000

_PARTS = (105600, 105600, 108800)
_OFFS = (0, 105600, 211200)
_NP = len(_PARTS)


def _ln_relu(h, ls, lb):
    mu = jnp.mean(h, axis=-1, keepdims=True)
    d = h - mu
    var = jnp.mean(d * d, axis=-1, keepdims=True)
    return jnp.maximum(d * lax.rsqrt(var + 1e-6) * ls + lb, 0.0)


def _dense(x, w, b, block_rows):
    rows, k = x.shape
    out_cols = w.shape[1]

    def body(x_ref, w_ref, b_ref, o_ref):
        o_ref[...] = jnp.dot(x_ref[...], w_ref[...],
                             preferred_element_type=F32) + b_ref[...]

    return pl.pallas_call(
        body,
        grid=(rows // block_rows,),
        in_specs=[pl.BlockSpec((block_rows, k), lambda i: (i, 0)),
                  pl.BlockSpec((k, out_cols), lambda i: (0, 0)),
                  pl.BlockSpec((1, out_cols), lambda i: (0, 0))],
        out_specs=pl.BlockSpec((block_rows, out_cols), lambda i: (i, 0)),
        out_shape=jax.ShapeDtypeStruct((rows, out_cols), F32),
    )(x, w, b)


def _proj(n, ew, eb, g, step):
    nb = N // _BN

    def body(n_ref, w_ref, eb_ref, g_ref, ps_ref, pr_ref, gv_ref):
        x = n_ref[...]
        ps_ref[...] = jnp.dot(x, w_ref[0, L:2 * L, :],
                              preferred_element_type=F32)
        pr_ref[...] = jnp.dot(x, w_ref[0, 2 * L:3 * L, :],
                              preferred_element_type=F32)

        @pl.when(pl.program_id(0) == 0)
        def _():
            gv_ref[...] = jnp.dot(g_ref[...], w_ref[0, 3 * L:4 * L, :],
                                  preferred_element_type=F32) + eb_ref[0]

    full = lambda i: (0, 0)
    stepw = lambda i: (step, 0, 0)
    stepb = lambda i: (step, 0, 0)
    return pl.pallas_call(
        body,
        grid=(nb,),
        in_specs=[pl.BlockSpec((_BN, L), lambda i: (i, 0)),
                  pl.BlockSpec((1, 4 * L, L), stepw),
                  pl.BlockSpec((1, 1, L), stepb),
                  pl.BlockSpec((1, L), full)],
        out_specs=[pl.BlockSpec((_BN, L), lambda i: (i, 0)),
                   pl.BlockSpec((_BN, L), lambda i: (i, 0)),
                   pl.BlockSpec((1, L), full)],
        out_shape=[jax.ShapeDtypeStruct((N, L), F32),
                   jax.ShapeDtypeStruct((N, L), F32),
                   jax.ShapeDtypeStruct((1, L), F32)],
    )(n, ew, eb, g)


def _edge_mlp(e, sum_rows, part, ew, gvec, els, elb, step):
    nb = _PARTS[part] // _BE
    off = _OFFS[part] // _BE

    def body(e_ref, s_ref, w_ref, gv_ref, ls_ref, lb_ref, o_ref, agg_ref):
        h = (jnp.dot(e_ref[...], w_ref[0, 0:L, :],
                     preferred_element_type=F32)
             + s_ref[...] + gv_ref[...])
        a = _ln_relu(h, ls_ref[0], lb_ref[0])
        o_ref[...] = a

        @pl.when(pl.program_id(0) == 0)
        def _():
            agg_ref[...] = jnp.zeros_like(agg_ref)

        agg_ref[...] += jnp.sum(a, axis=0, keepdims=True)

    full = lambda i: (0, 0)
    blk = lambda i: (i, 0)
    return pl.pallas_call(
        body,
        grid=(nb,),
        in_specs=[pl.BlockSpec((_BE, L), blk),
                  pl.BlockSpec((_BE, L), lambda i: (i + off, 0)),
                  pl.BlockSpec((1, 4 * L, L), lambda i: (step, 0, 0)),
                  pl.BlockSpec((1, L), full),
                  pl.BlockSpec((1, 1, L), lambda i: (step, 0, 0)),
                  pl.BlockSpec((1, 1, L), lambda i: (step, 0, 0))],
        out_specs=[pl.BlockSpec((_BE, L), blk),
                   pl.BlockSpec((1, L), full)],
        out_shape=[jax.ShapeDtypeStruct((_PARTS[part], L), F32),
                   jax.ShapeDtypeStruct((1, L), F32)],
    )(e, sum_rows, ew, gvec, els, elb)


def _edge_mlp0(efT, sum_rows, part, wemb, bemb, ew, gvec, els, elb):
    nb = _PARTS[part] // _BE
    off = _OFFS[part] // _BE

    def body(ef_ref, s_ref, wemb_ref, bemb_ref, w_ref, gv_ref,
             ls_ref, lb_ref, o_ref, agg_ref):
        t = lax.dot_general(ef_ref[...], wemb_ref[...],
                            (((0,), (0,)), ((), ())),
                            preferred_element_type=F32) + bemb_ref[...]
        h = (jnp.dot(t, w_ref[0, 0:L, :], preferred_element_type=F32)
             + s_ref[...] + gv_ref[...])
        a = _ln_relu(h, ls_ref[0], lb_ref[0])
        o_ref[...] = a

        @pl.when(pl.program_id(0) == 0)
        def _():
            agg_ref[...] = jnp.zeros_like(agg_ref)

        agg_ref[...] += jnp.sum(a, axis=0, keepdims=True)

    full = lambda i: (0, 0)
    blk = lambda i: (i, 0)
    return pl.pallas_call(
        body,
        grid=(nb,),
        in_specs=[pl.BlockSpec((DE, _BE), lambda i: (0, i + off)),
                  pl.BlockSpec((_BE, L), lambda i: (i + off, 0)),
                  pl.BlockSpec((DE, L), full),
                  pl.BlockSpec((1, L), full),
                  pl.BlockSpec((1, 4 * L, L), lambda i: (0, 0, 0)),
                  pl.BlockSpec((1, L), full),
                  pl.BlockSpec((1, 1, L), lambda i: (0, 0, 0)),
                  pl.BlockSpec((1, 1, L), lambda i: (0, 0, 0))],
        out_specs=[pl.BlockSpec((_BE, L), blk),
                   pl.BlockSpec((1, L), full)],
        out_shape=[jax.ShapeDtypeStruct((_PARTS[part], L), F32),
                   jax.ShapeDtypeStruct((1, L), F32)],
    )(efT, sum_rows, wemb, bemb, ew, gvec, els, elb)


def _node_glob(n, parts, nw, nb_, nls, nlb, g, eaggs, gw, gb, gls, glb, step):
    nblk = N // _BN

    def body(n_ref, p0_ref, p1_ref, p2_ref, q0_ref, q1_ref, q2_ref,
             w_ref, b_ref, ls_ref, lb_ref, g_ref, ea0_ref, ea1_ref, ea2_ref,
             gw_ref, gb_ref, gls_ref, glb_ref, o_ref, na_ref, go_ref):
        s = p0_ref[0] + p1_ref[0] + p2_ref[0]
        r = q0_ref[0] + q1_ref[0] + q2_ref[0]
        h = (jnp.dot(n_ref[...], w_ref[0, 0:L, :],
                     preferred_element_type=F32)
             + jnp.dot(s, w_ref[0, L:2 * L, :], preferred_element_type=F32)
             + jnp.dot(r, w_ref[0, 2 * L:3 * L, :],
                       preferred_element_type=F32)
             + jnp.dot(g_ref[...], w_ref[0, 3 * L:4 * L, :],
                       preferred_element_type=F32)
             + b_ref[0])
        a = _ln_relu(h, ls_ref[0], lb_ref[0])
        o_ref[...] = a

        @pl.when(pl.program_id(0) == 0)
        def _():
            na_ref[...] = jnp.zeros_like(na_ref)

        na_ref[...] += jnp.sum(a, axis=0, keepdims=True)

        @pl.when(pl.program_id(0) == nblk - 1)
        def _():
            ea = ea0_ref[...] + ea1_ref[...] + ea2_ref[...]
            hg = (jnp.dot(na_ref[...], gw_ref[0, 0:L, :],
                          preferred_element_type=F32)
                  + jnp.dot(ea, gw_ref[0, L:2 * L, :],
                            preferred_element_type=F32)
                  + jnp.dot(g_ref[...], gw_ref[0, 2 * L:3 * L, :],
                            preferred_element_type=F32)
                  + gb_ref[0])
            go_ref[...] = _ln_relu(hg, gls_ref[0], glb_ref[0])

    full = lambda i: (0, 0)
    blk = lambda i: (i, 0)
    stepb = lambda i: (step, 0, 0)
    return pl.pallas_call(
        body,
        grid=(nblk,),
        in_specs=[pl.BlockSpec((_BN, L), blk)]
        + [pl.BlockSpec((1, _BN, L), lambda i: (0, i, 0))] * 3
        + [pl.BlockSpec((1, _BN, L), lambda i: (1, i, 0))] * 3
        + [pl.BlockSpec((1, 4 * L, L), lambda i: (step, 0, 0)),
           pl.BlockSpec((1, 1, L), stepb),
           pl.BlockSpec((1, 1, L), stepb),
           pl.BlockSpec((1, 1, L), stepb),
           pl.BlockSpec((1, L), full),
           pl.BlockSpec((1, L), full),
           pl.BlockSpec((1, L), full),
           pl.BlockSpec((1, L), full),
           pl.BlockSpec((1, 3 * L, L), lambda i: (step, 0, 0)),
           pl.BlockSpec((1, 1, L), stepb),
           pl.BlockSpec((1, 1, L), stepb),
           pl.BlockSpec((1, 1, L), stepb)],
        out_specs=[pl.BlockSpec((_BN, L), blk),
                   pl.BlockSpec((1, L), full),
                   pl.BlockSpec((1, L), full)],
        out_shape=[jax.ShapeDtypeStruct((N, L), F32),
                   jax.ShapeDtypeStruct((1, L), F32),
                   jax.ShapeDtypeStruct((1, L), F32)],
    )(n, parts[0], parts[1], parts[2], parts[0], parts[1], parts[2],
      nw, nb_, nls, nlb, g, eaggs[0], eaggs[1], eaggs[2],
      gw, gb, gls, glb)


_EPW = E // NW
_CG = 80
_NBG = 5
_NGRP = _EPW // _CG // _NBG

_CS = 40
_NBS = 5

_MESH = plsc.VectorSubcoreMesh(core_axis_name="c", subcore_axis_name="s")


@functools.partial(
    pl.kernel,
    mesh=_MESH,
    out_type=jax.ShapeDtypeStruct((E, L), F32),
    scratch_types=[pltpu.VMEM((_EPW,), jnp.int32),
                   pltpu.VMEM((_EPW,), jnp.int32),
                   pltpu.VMEM((_NBG, _CG, L), F32),
                   pltpu.VMEM((_NBG, _CG, L), F32),
                   pltpu.SemaphoreType.DMA((_NBG,)),
                   pltpu.SemaphoreType.DMA((_NBG,)),
                   pltpu.SemaphoreType.DMA((_NBG,))],
)
def _sc_gather(ps_hbm, pr_hbm, snd_hbm, rcv_hbm, sum_out,
               idx_s, idx_r, rows_s, rows_r, sem_gs, sem_gr, sem_ws):
    wid = lax.axis_index("s") * NC + lax.axis_index("c")
    base0 = pl.multiple_of(wid * _EPW, 8)

    pltpu.sync_copy(snd_hbm.at[pl.ds(base0, _EPW)], idx_s)
    pltpu.sync_copy(rcv_hbm.at[pl.ds(base0, _EPW)], idx_r)

    def start_gather(b, ci):
        off = pl.multiple_of(ci * _CG, 8)
        pltpu.async_copy(ps_hbm.at[idx_s.at[pl.ds(off, _CG)]],
                         rows_s.at[b], sem_gs.at[b])
        pltpu.async_copy(pr_hbm.at[idx_r.at[pl.ds(off, _CG)]],
                         rows_r.at[b], sem_gr.at[b])

    for b in range(_NBG):
        start_gather(b, b)

    def group(g, carry):
        for b in range(_NBG):
            ci = g * _NBG + b
            base = pl.multiple_of(base0 + ci * _CG, 8)
            pltpu.make_async_copy(ps_hbm.at[idx_s.at[pl.ds(0, _CG)]],
                                  rows_s.at[b], sem_gs.at[b]).wait()
            pltpu.make_async_copy(pr_hbm.at[idx_r.at[pl.ds(0, _CG)]],
                                  rows_r.at[b], sem_gr.at[b]).wait()

            def add_row(r2, c2):
                for j in range(L // 16):
                    sl = pl.ds(j * 16, 16)
                    rows_s[b, r2, sl] = rows_s[b, r2, sl] + rows_r[b, r2, sl]
                return c2

            lax.fori_loop(0, _CG, add_row, 0)
            ws = pltpu.async_copy(rows_s.at[b], sum_out.at[pl.ds(base, _CG)],
                                  sem_ws.at[b])
            ws.wait()

            @pl.when(g < _NGRP - 1)
            def _():
                start_gather(b, ci + _NBG)
        return carry

    lax.fori_loop(0, _NGRP, group, 0)


def _make_scatter(ep):
    ept = ep // NS
    ngrps = ept // _CS // _NBS

    @functools.partial(
        pl.kernel,
        mesh=_MESH,
        out_type=jax.ShapeDtypeStruct((2, N, L), F32),
        scratch_types=[pltpu.VMEM((_NBS, _CS), jnp.int32),
                       pltpu.VMEM((_NBS, _CS, L), F32),
                       pltpu.VMEM_SHARED((N, L), F32),
                       pltpu.SemaphoreType.DMA((_NBS,)),
                       pltpu.SemaphoreType.DMA((_NBS,))],
    )
    def scatter(e_hbm, idx2_hbm, z_hbm, out_hbm,
                idx_b, rows_v, acc, sem_ld, sem_ix):
        cid = lax.axis_index("c")
        sid = lax.axis_index("s")

        @pl.when(sid == 0)
        def _():
            pltpu.sync_copy(z_hbm, acc)

        base0 = pl.multiple_of(sid * ept, 8)
        ibase0 = pl.multiple_of(cid * ep + sid * ept, 8)
        plsc.subcore_barrier()

        def start_load(b, ci):
            base = pl.multiple_of(base0 + ci * _CS, 8)
            ibase = pl.multiple_of(ibase0 + ci * _CS, 8)
            pltpu.async_copy(idx2_hbm.at[pl.ds(ibase, _CS)], idx_b.at[b],
                             sem_ix.at[b])
            pltpu.async_copy(e_hbm.at[pl.ds(base, _CS)], rows_v.at[b],
                             sem_ld.at[b])

        for b in range(_NBS):
            start_load(b, b)

        def group(g, carry):
            for b in range(_NBS):
                ci = g * _NBS + b
                pltpu.make_async_copy(idx2_hbm.at[pl.ds(0, _CS)],
                                      idx_b.at[b], sem_ix.at[b]).wait()
                pltpu.make_async_copy(e_hbm.at[pl.ds(0, _CS)],
                                      rows_v.at[b], sem_ld.at[b]).wait()
                pltpu.sync_copy(rows_v.at[b], acc.at[idx_b.at[b]], add=True)

                @pl.when(g < ngrps - 1)
                def _():
                    start_load(b, ci + _NBS)
            return carry

        lax.fori_loop(0, ngrps, group, 0)
        plsc.subcore_barrier()

        @pl.when(sid == 0)
        def _():
            pltpu.sync_copy(acc, out_hbm.at[cid])

    return scatter


_SCATTERS = {ep: _make_scatter(ep) for ep in set(_PARTS)}


def kernel(nodes, edge_feats, senders, receivers,
           embed_node_W, embed_node_b, embed_edge_W, embed_edge_b,
           edge_W, edge_b, edge_ls, edge_lb,
           node_W, node_b, node_ls, node_lb,
           glob_W, glob_b, glob_ls, glob_lb,
           dec_W, dec_b):
    steps = edge_W.shape[0]
    row = lambda v: v.reshape(1, -1)
    r3 = lambda v: v.reshape(steps, 1, -1)
    edge_b, edge_ls, edge_lb = r3(edge_b), r3(edge_ls), r3(edge_lb)
    node_b, node_ls, node_lb = r3(node_b), r3(node_ls), r3(node_lb)
    glob_b, glob_ls, glob_lb = r3(glob_b), r3(glob_ls), r3(glob_lb)

    n = _dense(nodes, embed_node_W, row(embed_node_b), _BN)
    efT = jnp.transpose(edge_feats)
    g = jnp.zeros((1, L), F32)
    zeros_n = jnp.zeros((N, L), F32)
    snd = senders.astype(jnp.int32)
    rcv = receivers.astype(jnp.int32)
    idx2p = tuple(jnp.concatenate([snd[o:o + s], rcv[o:o + s]])
                  for s, o in zip(_PARTS, _OFFS))

    e = [None] * _NP
    for i in range(steps):
        ps, pr, gvec = _proj(n, edge_W, edge_b, g, i)
        sum_rows = _sc_gather(ps, pr, snd, rcv)
        eaggs = [None] * _NP
        parts = [None] * _NP
        for p in range(_NP):
            if i == 0:
                e[p], eaggs[p] = _edge_mlp0(
                    efT, sum_rows, p, embed_edge_W, row(embed_edge_b),
                    edge_W, gvec, edge_ls, edge_lb)
            else:
                e[p], eaggs[p] = _edge_mlp(
                    e[p], sum_rows, p, edge_W, gvec, edge_ls, edge_lb, i)
            parts[p] = _SCATTERS[_PARTS[p]](e[p], idx2p[p], zeros_n)
        n, _nagg, g = _node_glob(
            n, parts, node_W, node_b, node_ls, node_lb,
            g, eaggs, glob_W, glob_b, glob_ls, glob_lb, i)

    return _dense(g, dec_W, row(dec_b), 1)
